# Initial kernel scaffold; baseline (speedup 1.0000x reference)
import jax
import jax.numpy as jnp
from jax.experimental import pallas as pl


def kernel(x, edge_index, W1_l, W1_r, b1, W2_l, W2_r, b2):
    raise NotImplementedError("write your pallas kernel here")


# speedup vs baseline: 5.2381x; 5.2381x over previous
import dataclasses
import functools

import jax
import jax.numpy as jnp
from jax import lax
from jax.experimental import pallas as pl
from jax.experimental.pallas import tpu as pltpu
from jax.experimental.pallas import tpu_sc as plsc

N = 10000
E = 160000
D = 256
H = D // 2
NC = 2
NS = 16
EPT = E // NS
CH = 80
NCHUNK = EPT // CH
SCH = 25
STG = NCHUNK // SCH
NP_ = 10240
STRIPE = NP_ // NS
MB = 1024


def _sc_mesh():
    return plsc.VectorSubcoreMesh(
        core_axis_name="c", subcore_axis_name="s", num_cores=NC, num_subcores=NS
    )


def _sc_params():
    cp = pltpu.CompilerParams()
    if "needs_layout_passes" in pltpu.CompilerParams.__dataclass_fields__:
        cp = dataclasses.replace(cp, needs_layout_passes=False)
    return cp


def _make_sc_agg(with_deg):
    agg_t = jax.ShapeDtypeStruct((NC, NP_, H), jnp.float32)
    deg_t = jax.ShapeDtypeStruct((NS, NP_), jnp.float32)
    scratch = [
        pltpu.VMEM((SCH, CH), jnp.int32),
        pltpu.VMEM((SCH, CH), jnp.int32),
        pltpu.VMEM((CH, H), jnp.float32),
        pltpu.VMEM_SHARED((NP_, H), jnp.float32),
        pltpu.SemaphoreType.DMA,
    ]
    if with_deg:
        scratch.insert(3, pltpu.VMEM((NP_,), jnp.float32))

    def body(xl, srcr, dstr, zrow, agg, deg,
             src_v, dst_v, rows_v, hist, acc_sh, sem):
        c = lax.axis_index("c")
        s = lax.axis_index("s")
        pltpu.sync_copy(zrow, acc_sh.at[pl.ds(s * STRIPE, STRIPE)])
        if with_deg:
            @pl.when(c == 0)
            def _():
                @pl.loop(0, NP_ // 16)
                def _(i):
                    hist[pl.ds(16 * i, 16)] = jnp.zeros((16,), jnp.float32)
        plsc.subcore_barrier()

        @pl.loop(0, STG)
        def _(t):
            pltpu.sync_copy(srcr.at[s].at[t], src_v)
            pltpu.sync_copy(dstr.at[s].at[t], dst_v)

            @pl.loop(0, SCH)
            def _(i):
                pltpu.async_copy(xl.at[c].at[src_v.at[i]], rows_v, sem).wait()
                pltpu.sync_copy(rows_v, acc_sh.at[dst_v.at[i]], add=True)

            if with_deg:
                @pl.when(c == 0)
                def _():
                    @pl.loop(0, SCH)
                    def _(i):
                        @pl.loop(0, CH // 16)
                        def _(j):
                            vec = dst_v[i, pl.ds(16 * j, 16)]
                            plsc.addupdate_scatter(
                                hist, [vec], jnp.full((16,), 1.0, jnp.float32))

        plsc.subcore_barrier()
        rows = pl.ds(s * STRIPE, STRIPE)
        pltpu.sync_copy(acc_sh.at[rows], agg.at[c].at[rows])
        if with_deg:
            @pl.when(c == 0)
            def _():
                pltpu.sync_copy(hist, deg.at[s])

    if with_deg:
        @functools.partial(
            pl.kernel, out_type=(agg_t, deg_t), mesh=_sc_mesh(),
            scratch_types=scratch, compiler_params=_sc_params(),
        )
        def sc_agg(xl, srcr, dstr, zrow, agg, deg,
                   src_v, dst_v, rows_v, hist, acc_sh, sem):
            body(xl, srcr, dstr, zrow, agg, deg,
                 src_v, dst_v, rows_v, hist, acc_sh, sem)
    else:
        @functools.partial(
            pl.kernel, out_type=agg_t, mesh=_sc_mesh(),
            scratch_types=scratch, compiler_params=_sc_params(),
        )
        def sc_agg(xl, srcr, dstr, zrow, agg,
                   src_v, dst_v, rows_v, acc_sh, sem):
            body(xl, srcr, dstr, zrow, agg, None,
                 src_v, dst_v, rows_v, None, acc_sh, sem)

    return sc_agg


_sc_agg_deg = _make_sc_agg(True)
_sc_agg_only = _make_sc_agg(False)


def _mm_in(x, W_l, W_r, b):
    def tc_body(x_ref, wl_ref, wr_ref, b_ref, xl_ref, xr_ref):
        xb = x_ref[...]
        l = jnp.dot(xb, wl_ref[...], preferred_element_type=jnp.float32)
        xl_ref[0] = l[:, :H]
        xl_ref[1] = l[:, H:]
        xr_ref[...] = (
            jnp.dot(xb, wr_ref[...], preferred_element_type=jnp.float32)
            + b_ref[...]
        )

    return pl.pallas_call(
        tc_body,
        grid=(pl.cdiv(N, MB),),
        in_specs=[
            pl.BlockSpec((MB, D), lambda i: (i, 0)),
            pl.BlockSpec((D, D), lambda i: (0, 0)),
            pl.BlockSpec((D, D), lambda i: (0, 0)),
            pl.BlockSpec((1, D), lambda i: (0, 0)),
        ],
        out_specs=[
            pl.BlockSpec((NC, MB, H), lambda i: (0, i, 0)),
            pl.BlockSpec((MB, D), lambda i: (i, 0)),
        ],
        out_shape=[
            jax.ShapeDtypeStruct((NC, N, H), jnp.float32),
            jax.ShapeDtypeStruct((N, D), jnp.float32),
        ],
    )(x, W_l, W_r, b.reshape(1, D))


def _mm_mid(agg, deg16, xr, W_l, W_r, b):
    def tc_body(agg_ref, deg_ref, xr_ref, wl_ref, wr_ref, b_ref,
                hl_ref, hr_ref):
        deg = jnp.maximum(jnp.sum(deg_ref[...], axis=0), 1.0).reshape(MB, 1)
        mean = jnp.concatenate([agg_ref[0], agg_ref[1]], axis=1) / deg
        h = jnp.maximum(mean + xr_ref[...], 0.0)
        l = jnp.dot(h, wl_ref[...], preferred_element_type=jnp.float32)
        hl_ref[0] = l[:, :H]
        hl_ref[1] = l[:, H:]
        hr_ref[...] = (
            jnp.dot(h, wr_ref[...], preferred_element_type=jnp.float32)
            + b_ref[...]
        )

    return pl.pallas_call(
        tc_body,
        grid=(pl.cdiv(N, MB),),
        in_specs=[
            pl.BlockSpec((NC, MB, H), lambda i: (0, i, 0)),
            pl.BlockSpec((NS, MB), lambda i: (0, i)),
            pl.BlockSpec((MB, D), lambda i: (i, 0)),
            pl.BlockSpec((D, D), lambda i: (0, 0)),
            pl.BlockSpec((D, D), lambda i: (0, 0)),
            pl.BlockSpec((1, D), lambda i: (0, 0)),
        ],
        out_specs=[
            pl.BlockSpec((NC, MB, H), lambda i: (0, i, 0)),
            pl.BlockSpec((MB, D), lambda i: (i, 0)),
        ],
        out_shape=[
            jax.ShapeDtypeStruct((NC, N, H), jnp.float32),
            jax.ShapeDtypeStruct((N, D), jnp.float32),
        ],
    )(agg, deg16, xr, W_l, W_r, b.reshape(1, D))


def _mm_out(agg, deg16, hr):
    def tc_body(agg_ref, deg_ref, hr_ref, o_ref):
        deg = jnp.maximum(jnp.sum(deg_ref[...], axis=0), 1.0).reshape(MB, 1)
        mean = jnp.concatenate([agg_ref[0], agg_ref[1]], axis=1) / deg
        o_ref[...] = mean + hr_ref[...]

    return pl.pallas_call(
        tc_body,
        grid=(pl.cdiv(N, MB),),
        in_specs=[
            pl.BlockSpec((NC, MB, H), lambda i: (0, i, 0)),
            pl.BlockSpec((NS, MB), lambda i: (0, i)),
            pl.BlockSpec((MB, D), lambda i: (i, 0)),
        ],
        out_specs=pl.BlockSpec((MB, D), lambda i: (i, 0)),
        out_shape=jax.ShapeDtypeStruct((N, D), jnp.float32),
    )(agg, deg16, hr)


def kernel(x, edge_index, W1_l, W1_r, b1, W2_l, W2_r, b2):
    src = edge_index[0].astype(jnp.int32).reshape(NS, STG, SCH, CH)
    dst = edge_index[1].astype(jnp.int32).reshape(NS, STG, SCH, CH)
    zrow = jnp.zeros((STRIPE, H), jnp.float32)

    xl, xr1 = _mm_in(x, W1_l, W1_r, b1)
    agg1, deg = _sc_agg_deg(xl, src, dst, zrow)
    hl, hr2 = _mm_mid(agg1, deg, xr1, W2_l, W2_r, b2)
    agg2 = _sc_agg_only(hl, src, dst, zrow)
    return _mm_out(agg2, deg, hr2)
